# Initial kernel scaffold; baseline (speedup 1.0000x reference)
import jax
import jax.numpy as jnp
from jax.experimental import pallas as pl


def kernel(edge_index, edge_feats, sink_node_mask, W1, b1, W2, b2):
    raise NotImplementedError("write your pallas kernel here")


# speedup vs baseline: 50.6412x; 50.6412x over previous
import functools

import jax
import jax.numpy as jnp
from jax import lax
from jax.experimental import pallas as pl
from jax.experimental.pallas import tpu as pltpu
from jax.experimental.pallas import tpu_sc as plsc

_N = 10000
_E = 160000
_D = 256
_ITERS = 30
_LANES = 128
_ROWS = 80
_NPAD = _ROWS * _LANES
_NSC = 16
_TILE_E = _E // _NSC
_GROUPS = _TILE_E // 16
_RPT = _ROWS // _NSC
_CHUNK = _NPAD // _NSC

_MLP_BLOCK = 8000


def _dot3(a, b):
    ah = a.astype(jnp.bfloat16)
    al = (a - ah.astype(jnp.float32)).astype(jnp.bfloat16)
    bh = b.astype(jnp.bfloat16)
    bl = (b - bh.astype(jnp.float32)).astype(jnp.bfloat16)
    dn = (((1,), (0,)), ((), ()))
    o = lax.dot_general(ah, bh, dn, preferred_element_type=jnp.float32)
    o = o + lax.dot_general(ah, bl, dn, preferred_element_type=jnp.float32)
    o = o + lax.dot_general(al, bh, dn, preferred_element_type=jnp.float32)
    return o


def _dot3t(a, b):
    ah = a.astype(jnp.bfloat16)
    al = (a - ah.astype(jnp.float32)).astype(jnp.bfloat16)
    bh = b.astype(jnp.bfloat16)
    bl = (b - bh.astype(jnp.float32)).astype(jnp.bfloat16)
    dn = (((1,), (1,)), ((), ()))
    o = lax.dot_general(ah, bh, dn, preferred_element_type=jnp.float32)
    o = o + lax.dot_general(ah, bl, dn, preferred_element_type=jnp.float32)
    o = o + lax.dot_general(al, bh, dn, preferred_element_type=jnp.float32)
    return o


def _mlp_body(efh_ref, efl_ref, w1_ref, b1_ref, w2t_ref, b2_ref,
              rew_ref, er_ref):
    xh = efh_ref[...]
    xl = efl_ref[...]
    w1 = w1_ref[...]
    w1h = w1.astype(jnp.bfloat16)
    w1l = (w1 - w1h.astype(jnp.float32)).astype(jnp.bfloat16)
    dn = (((1,), (0,)), ((), ()))
    hA = lax.dot_general(xh, w1h, dn, preferred_element_type=jnp.float32)
    hB = lax.dot_general(xh, w1l, dn, preferred_element_type=jnp.float32)
    hC = lax.dot_general(xl, w1h, dn, preferred_element_type=jnp.float32)
    h = jnp.maximum((hA + hB) + (hC + b1_ref[...]), 0.0)
    logits = _dot3t(w2t_ref[...], h)
    logits = logits + b2_ref[...]
    sp = jnp.maximum(logits, 0.0) + jnp.log1p(jnp.exp(-jnp.abs(logits)))
    rew_ref[0] = -sp
    er_ref[0] = jnp.exp(-sp)


def _encoder(edge_feats, W1, b1, W2, b2):
    grid = _E // _MLP_BLOCK
    rew2, er2 = pl.pallas_call(
        _mlp_body,
        grid=(grid,),
        in_specs=[
            pl.BlockSpec((_MLP_BLOCK, _D), lambda i: (i, 0)),
            pl.BlockSpec((_MLP_BLOCK, _D), lambda i: (i, 0)),
            pl.BlockSpec((_D, _D), lambda i: (0, 0)),
            pl.BlockSpec((1, _D), lambda i: (0, 0)),
            pl.BlockSpec((1, _D), lambda i: (0, 0)),
            pl.BlockSpec((1, 1), lambda i: (0, 0)),
        ],
        out_specs=[
            pl.BlockSpec((1, 1, _MLP_BLOCK), lambda i: (i, 0, 0)),
            pl.BlockSpec((1, 1, _MLP_BLOCK), lambda i: (i, 0, 0)),
        ],
        out_shape=[
            jax.ShapeDtypeStruct((grid, 1, _MLP_BLOCK), jnp.float32),
            jax.ShapeDtypeStruct((grid, 1, _MLP_BLOCK), jnp.float32),
        ],
    )(edge_feats.astype(jnp.bfloat16),
      (edge_feats - edge_feats.astype(jnp.bfloat16).astype(jnp.float32)
       ).astype(jnp.bfloat16),
      W1, b1.reshape(1, _D), W2.reshape(1, _D), b2.reshape(1, 1))
    return rew2.reshape(_E), er2.reshape(_E)


def _fp_body(dst_hbm, src_hbm, er_hbm, sinkf_hbm, x_out, ep_out,
             dst_v, src_v, er_v, x_v, acc_v, sfc_v, red_v, xc_v, ep_v,
             shared_all, shared_x, sem):
    c = lax.axis_index("c")
    s = lax.axis_index("s")
    ebase = s * _TILE_E
    nbase = s * _CHUNK

    pltpu.sync_copy(dst_hbm.at[pl.ds(ebase, _TILE_E)], dst_v)
    pltpu.sync_copy(src_hbm.at[pl.ds(ebase, _TILE_E)], src_v)
    pltpu.sync_copy(er_hbm.at[pl.ds(ebase, _TILE_E)], er_v)
    pltpu.sync_copy(sinkf_hbm, x_v)
    pltpu.sync_copy(sinkf_hbm.at[pl.ds(nbase, _CHUNK)], sfc_v)

    zeros16 = jnp.zeros((16,), jnp.float32)

    def _zero_acc(i, _):
        acc_v[pl.ds(i * 16, 16)] = zeros16
        return 0

    lax.fori_loop(0, _NPAD // 16, _zero_acc, 0, unroll=8)

    def _iter(_, carry):
        def _edge(j, _):
            off = j * 16
            d16 = dst_v[pl.ds(off, 16)]
            s16 = src_v[pl.ds(off, 16)]
            e16 = er_v[pl.ds(off, 16)]
            xv = plsc.load_gather(x_v, [d16])
            plsc.addupdate_scatter(acc_v, [s16], e16 * xv)
            return 0

        lax.fori_loop(0, _GROUPS, _edge, 0, unroll=25)

        pltpu.sync_copy(acc_v, shared_all.at[s])
        plsc.subcore_barrier()

        pltpu.sync_copy(shared_all.at[:, pl.ds(nbase, _CHUNK)], red_v)

        def _red(g, _):
            col = g * 16
            v = red_v[0, pl.ds(col, 16)]
            for r in range(1, _NSC):
                v = v + red_v[r, pl.ds(col, 16)]
            sf = sfc_v[pl.ds(col, 16)]
            xc_v[pl.ds(col, 16)] = sf + (1.0 - sf) * v
            return 0

        lax.fori_loop(0, _CHUNK // 16, _red, 0, unroll=2)

        pltpu.sync_copy(xc_v, shared_x.at[pl.ds(nbase, _CHUNK)])
        plsc.subcore_barrier()

        cp = pltpu.async_copy(shared_x, x_v, sem)
        lax.fori_loop(0, _NPAD // 16, _zero_acc, 0, unroll=8)
        cp.wait()
        plsc.subcore_barrier()
        return carry

    lax.fori_loop(0, _ITERS, _iter, 0)

    def _ep(j, _):
        off = j * 16
        d16 = dst_v[pl.ds(off, 16)]
        s16 = src_v[pl.ds(off, 16)]
        e16 = er_v[pl.ds(off, 16)]
        xd = plsc.load_gather(x_v, [d16])
        xs = plsc.load_gather(x_v, [s16])
        ep_v[pl.ds(off, 16)] = e16 * xd / jnp.maximum(xs, 1e-30)
        return 0

    lax.fori_loop(0, _GROUPS, _ep, 0, unroll=25)

    @pl.when(c == 0)
    def _():
        pltpu.sync_copy(ep_v, ep_out.at[pl.ds(ebase, _TILE_E)])

    @pl.when(jnp.logical_and(c == 0, s == 0))
    def _():
        pltpu.sync_copy(x_v, x_out)


_fixed_point = pl.kernel(
    _fp_body,
    out_type=[
        jax.ShapeDtypeStruct((_NPAD,), jnp.float32),
        jax.ShapeDtypeStruct((_E,), jnp.float32),
    ],
    mesh=plsc.VectorSubcoreMesh(core_axis_name="c", subcore_axis_name="s"),
    compiler_params=pltpu.CompilerParams(needs_layout_passes=False),
    scratch_types=[
        pltpu.VMEM((_TILE_E,), jnp.int32),
        pltpu.VMEM((_TILE_E,), jnp.int32),
        pltpu.VMEM((_TILE_E,), jnp.float32),
        pltpu.VMEM((_NPAD,), jnp.float32),
        pltpu.VMEM((_NPAD,), jnp.float32),
        pltpu.VMEM((_CHUNK,), jnp.float32),
        pltpu.VMEM((_NSC, _CHUNK), jnp.float32),
        pltpu.VMEM((_CHUNK,), jnp.float32),
        pltpu.VMEM((_TILE_E,), jnp.float32),
        pltpu.VMEM_SHARED((_NSC, _NPAD), jnp.float32),
        pltpu.VMEM_SHARED((_NPAD,), jnp.float32),
        pltpu.SemaphoreType.DMA,
    ],
)


def _log_body(x_ref, v_ref):
    v_ref[...] = jnp.log(jnp.maximum(x_ref[...], 1e-30))


def _log_values(x2d):
    return pl.pallas_call(
        _log_body,
        out_shape=jax.ShapeDtypeStruct((_ROWS, _LANES), jnp.float32),
    )(x2d)


def kernel(edge_index, edge_feats, sink_node_mask, W1, b1, W2, b2):
    src = edge_index[0]
    dst = edge_index[1]
    sinkf2d = jnp.pad(sink_node_mask.astype(jnp.float32),
                      (0, _NPAD - _N)).reshape(_ROWS, _LANES)
    rewards, exp_rewards = _encoder(edge_feats, W1, b1, W2, b2)
    x1d, edge_probs = _fixed_point(dst, src, exp_rewards, sinkf2d.reshape(_NPAD))
    values = _log_values(x1d.reshape(_ROWS, _LANES)).reshape(_NPAD)[:_N]
    return rewards, values, edge_probs
